# baseline (speedup 1.0000x reference)
import functools

import jax
import jax.numpy as jnp
from jax import lax
from jax.experimental import pallas as pl
from jax.experimental.pallas import tpu as pltpu
from jax.experimental.pallas import tpu_sc as plsc

B = 16384
D = 32
L = 16
NC, NS = 2, 16
NW = NC * NS
BPW = B // NW
CHUNK = 128
NCHUNK = BPW // CHUNK
GROUPS = BPW // L

_mesh = plsc.VectorSubcoreMesh(core_axis_name="c", subcore_axis_name="s")


@functools.partial(
    pl.kernel,
    mesh=_mesh,
    compiler_params=pltpu.CompilerParams(
        needs_layout_passes=False, use_tc_tiling_on_sc=False),
    out_type=(
        jax.ShapeDtypeStruct((B,), jnp.float32),
        jax.ShapeDtypeStruct((B,), jnp.float32),
    ),
    scratch_types=[
        pltpu.VMEM((NCHUNK, CHUNK), jnp.int32),
        pltpu.VMEM((NCHUNK, CHUNK), jnp.int32),
        pltpu.VMEM((NCHUNK, CHUNK), jnp.int32),
        pltpu.VMEM((BPW, D), jnp.float32),
        pltpu.VMEM((BPW, D), jnp.float32),
        pltpu.VMEM((BPW, D), jnp.float32),
        pltpu.VMEM((BPW,), jnp.float32),
        pltpu.VMEM((BPW,), jnp.float32),
        pltpu.SemaphoreType.DMA,
    ],
)
def _mf_kernel(users_hbm, pos_hbm, neg_hbm, utab_hbm, itab_hbm,
               pout_hbm, nout_hbm,
               uidx, pidx, nidx, urows, prows, nrows, pout, nout, sem):
    wid = lax.axis_index("s") * NC + lax.axis_index("c")
    base = wid * BPW
    cbase = wid * NCHUNK

    pltpu.sync_copy(users_hbm.at[pl.ds(cbase, NCHUNK)], uidx)
    pltpu.sync_copy(pos_hbm.at[pl.ds(cbase, NCHUNK)], pidx)
    pltpu.sync_copy(neg_hbm.at[pl.ds(cbase, NCHUNK)], nidx)

    copies = []
    for idx_ref, tab, rows in ((uidx, utab_hbm, urows),
                               (pidx, itab_hbm, prows),
                               (nidx, itab_hbm, nrows)):
        for c in range(NCHUNK):
            copies.append(
                pltpu.async_copy(tab.at[idx_ref.at[c]],
                                 rows.at[pl.ds(c * CHUNK, CHUNK)], sem))
    for cp in copies:
        cp.wait()

    def group(g, carry):
        row0 = g * L
        ridx = row0 + lax.iota(jnp.int32, L)
        accp = jnp.zeros((L,), jnp.float32)
        accn = jnp.zeros((L,), jnp.float32)
        for d in range(D):
            cidx = jnp.full((L,), d, jnp.int32)
            uv = plsc.load_gather(urows, [ridx, cidx])
            pv = plsc.load_gather(prows, [ridx, cidx])
            nv = plsc.load_gather(nrows, [ridx, cidx])
            accp = accp + uv * pv
            accn = accn + uv * nv
        pout[pl.ds(row0, L)] = accp
        nout[pl.ds(row0, L)] = accn
        return carry

    lax.fori_loop(0, GROUPS, group, 0)

    pltpu.sync_copy(pout, pout_hbm.at[pl.ds(base, BPW)])
    pltpu.sync_copy(nout, nout_hbm.at[pl.ds(base, BPW)])


def kernel(users, positive_items, negative_items, user_table, item_table):
    u = users.astype(jnp.int32).reshape(NW * NCHUNK, CHUNK)
    p = positive_items.astype(jnp.int32).reshape(NW * NCHUNK, CHUNK)
    n = negative_items.astype(jnp.int32).reshape(NW * NCHUNK, CHUNK)
    return _mf_kernel(u, p, n, user_table, item_table)
